# baseline (speedup 1.0000x reference)
import numpy as np
import jax
import jax.numpy as jnp
from jax.experimental import pallas as pl
from jax.experimental.pallas import tpu as pltpu

_N = 128
_BATCH_TILE = 16384
_NCHUNK = 8
_CLEN = 32
_GPAD = _NCHUNK * _CLEN
_TROWS = 64


def _static_grids():
    list_wires = [(j - 1, j) for i in range(1, _N) for j in range(i, max(0, i - _N), -1)]
    pos = {}
    groups = [[]]
    for wires in list_wires:
        g_pos = max(pos.get(w, 0) for w in wires)
        while len(groups) - 1 < g_pos:
            groups.append([])
        groups[g_pos].append(wires)
        for w in wires:
            pos[w] = g_pos + 1
    G = len(groups)
    offs = np.zeros((_GPAD,), np.int32)
    act = np.zeros((_GPAD, _N), np.float32)
    left = np.zeros((_GPAD, _N), np.float32)
    layer_off = np.zeros((G,), np.int64)
    layer_pairs = [None] * G
    off = 0
    for g, grp in enumerate(groups):
        layer_off[g] = off
        layer_pairs[g] = grp
        off += len(grp)
    for c in range(_NCHUNK):
        for i in range(_CLEN):
            g = _CLEN * c + i
            r = _NCHUNK * i + c
            if g >= G:
                continue
            offs[r] = max(2 * layer_off[g] - (g % 2), 0)
            for (p0, p1) in layer_pairs[g]:
                act[r, p0] = 1.0
                act[r, p1] = 1.0
                left[r, p0] = 1.0
    return G, offs, act, left


_G, _OFFS, _ACT, _LEFT = _static_grids()


def _kernel_body(offs_ref, th_ref, act_ref, left_ref, x_ref, bias_ref, out_ref,
                 th_scr, w_scr, cs_scr, cl_scr, cr_scr, qa_scr, qb_scr, q_scr):
    @pl.when(pl.program_id(0) == 0)
    def _build_q():
        lane2 = jax.lax.broadcasted_iota(jnp.int32, (1, _N), 1)

        th64 = th_ref[:, :]
        b = th64.reshape(_TROWS, 2, 64).reshape(2 * _TROWS, 64)
        c = jnp.swapaxes(b, 0, 1)
        c2 = jnp.broadcast_to(
            c.reshape(64, 1, 2 * _TROWS), (64, 2, 2 * _TROWS)
        ).reshape(_N, 2 * _TROWS)
        th_scr[:, :] = jnp.swapaxes(c2, 0, 1)

        def window_row(r):
            off = offs_ref[r]
            r0 = jax.lax.shift_right_logical(off, 7)
            l0 = jax.lax.bitwise_and(off, 127)
            two = th_scr[pl.ds(r0, 2), :]
            rolled = pltpu.roll(two, 128 - l0, 1)
            return jnp.where(lane2 < 128 - l0, rolled[0:1, :], rolled[1:2, :])

        def window_body(u, carry):
            r = 4 * u
            w_scr[pl.ds(r, 4), :] = jnp.concatenate(
                [window_row(r), window_row(r + 1),
                 window_row(r + 2), window_row(r + 3)], axis=0)
            return carry

        jax.lax.fori_loop(0, _GPAD // 4, window_body, 0)

        c_grid = jnp.cos(w_scr[:, :])
        s_grid = jnp.sin(w_scr[:, :])
        a = act_ref[:, :]
        l = left_ref[:, :]
        cs_scr[:, :] = a * c_grid + (1.0 - a)
        cl_scr[:, :] = -l * s_grid
        cr_scr[:, :] = (a - l) * s_grid

        rows = jax.lax.broadcasted_iota(jnp.int32, (_NCHUNK, _N, _N), 1)
        cols = jax.lax.broadcasted_iota(jnp.int32, (_NCHUNK, _N, _N), 2)
        qa_scr[:, :, :] = (rows == cols).astype(jnp.float32)

        def apply_layer(q, i):
            coef_self = cs_scr[pl.ds(_NCHUNK * i, _NCHUNK), :].reshape(_NCHUNK, 1, _N)
            coef_l = cl_scr[pl.ds(_NCHUNK * i, _NCHUNK), :].reshape(_NCHUNK, 1, _N)
            coef_r = cr_scr[pl.ds(_NCHUNK * i, _NCHUNK), :].reshape(_NCHUNK, 1, _N)
            q_left = pltpu.roll(q, _N - 1, 2)
            q_right = pltpu.roll(q, 1, 2)
            return coef_self * q + coef_l * q_left + coef_r * q_right

        def body(u, carry):
            q = qa_scr[:, :, :]
            q = apply_layer(q, 2 * u)
            q = apply_layer(q, 2 * u + 1)
            qa_scr[:, :, :] = q
            return carry

        jax.lax.fori_loop(0, _CLEN // 2, body, 0)

        hi = jax.lax.Precision.HIGHEST
        for j in range(4):
            qb_scr[j, :, :] = jnp.dot(
                qa_scr[2 * j, :, :], qa_scr[2 * j + 1, :, :],
                precision=hi, preferred_element_type=jnp.float32)
        for j in range(2):
            qa_scr[j, :, :] = jnp.dot(
                qb_scr[2 * j, :, :], qb_scr[2 * j + 1, :, :],
                precision=hi, preferred_element_type=jnp.float32)
        q_scr[:, :] = jnp.dot(
            qa_scr[0, :, :], qa_scr[1, :, :],
            precision=hi, preferred_element_type=jnp.float32)

    out_ref[:, :] = (
        jnp.dot(x_ref[:, :], q_scr[:, :], preferred_element_type=jnp.float32)
        + bias_ref[0:1, :]
    )


def kernel(inputs, thetas, bias):
    batch = inputs.shape[0]
    th_pad = jnp.pad(thetas, (0, _TROWS * _N - thetas.shape[0])).reshape(_TROWS, _N)
    bias2d = bias.reshape(1, _N)
    grid = batch // _BATCH_TILE
    return pl.pallas_call(
        _kernel_body,
        out_shape=jax.ShapeDtypeStruct((batch, _N), jnp.float32),
        grid=(grid,),
        in_specs=[
            pl.BlockSpec(memory_space=pltpu.SMEM),
            pl.BlockSpec((_TROWS, _N), lambda i: (0, 0)),
            pl.BlockSpec((_GPAD, _N), lambda i: (0, 0)),
            pl.BlockSpec((_GPAD, _N), lambda i: (0, 0)),
            pl.BlockSpec((_BATCH_TILE, _N), lambda i: (i, 0)),
            pl.BlockSpec((1, _N), lambda i: (0, 0)),
        ],
        out_specs=pl.BlockSpec((_BATCH_TILE, _N), lambda i: (i, 0)),
        scratch_shapes=[
            pltpu.VMEM((_N, _N), jnp.float32),
            pltpu.VMEM((_GPAD, _N), jnp.float32),
            pltpu.VMEM((_GPAD, _N), jnp.float32),
            pltpu.VMEM((_GPAD, _N), jnp.float32),
            pltpu.VMEM((_GPAD, _N), jnp.float32),
            pltpu.VMEM((_NCHUNK, _N, _N), jnp.float32),
            pltpu.VMEM((_NCHUNK // 2, _N, _N), jnp.float32),
            pltpu.VMEM((_N, _N), jnp.float32),
        ],
        compiler_params=pltpu.CompilerParams(
            dimension_semantics=("arbitrary",),
        ),
    )(jnp.asarray(_OFFS), th_pad, jnp.asarray(_ACT), jnp.asarray(_LEFT),
      inputs, bias2d)


# speedup vs baseline: 1.1937x; 1.1937x over previous
import numpy as np
import jax
import jax.numpy as jnp
from jax.experimental import pallas as pl
from jax.experimental.pallas import tpu as pltpu

_N = 128
_BATCH_TILE = 8192
_NCHUNK = 8
_CLEN = 32
_GPAD = _NCHUNK * _CLEN
_TROWS = 64


def _static_grids():
    list_wires = [(j - 1, j) for i in range(1, _N) for j in range(i, max(0, i - _N), -1)]
    pos = {}
    groups = [[]]
    for wires in list_wires:
        g_pos = max(pos.get(w, 0) for w in wires)
        while len(groups) - 1 < g_pos:
            groups.append([])
        groups[g_pos].append(wires)
        for w in wires:
            pos[w] = g_pos + 1
    G = len(groups)
    offs = np.zeros((_GPAD,), np.int32)
    act = np.zeros((_GPAD, _N), np.float32)
    left = np.zeros((_GPAD, _N), np.float32)
    layer_off = np.zeros((G,), np.int64)
    layer_pairs = [None] * G
    off = 0
    for g, grp in enumerate(groups):
        layer_off[g] = off
        layer_pairs[g] = grp
        off += len(grp)
    for c in range(_NCHUNK):
        for i in range(_CLEN):
            g = _CLEN * c + i
            r = _NCHUNK * i + c
            if g >= G:
                continue
            offs[r] = max(2 * layer_off[g] - (g % 2), 0)
            for (p0, p1) in layer_pairs[g]:
                act[r, p0] = 1.0
                act[r, p1] = 1.0
                left[r, p0] = 1.0
    return G, offs, act, left


_G, _OFFS, _ACT, _LEFT = _static_grids()


def _kernel_body(offs_ref, th_ref, act_ref, left_ref, x_ref, bias_ref, out_ref,
                 th_scr, w_scr, cs_scr, cl_scr, cr_scr, qa_scr, qb_scr, q_scr):
    @pl.when(pl.program_id(0) == 0)
    def _build_q():
        lane2 = jax.lax.broadcasted_iota(jnp.int32, (1, _N), 1)

        th64 = th_ref[:, :]
        b = th64.reshape(_TROWS, 2, 64).reshape(2 * _TROWS, 64)
        c = jnp.swapaxes(b, 0, 1)
        c2 = jnp.broadcast_to(
            c.reshape(64, 1, 2 * _TROWS), (64, 2, 2 * _TROWS)
        ).reshape(_N, 2 * _TROWS)
        th_scr[:, :] = jnp.swapaxes(c2, 0, 1)

        def window_row(r):
            off = offs_ref[r]
            r0 = jax.lax.shift_right_logical(off, 7)
            l0 = jax.lax.bitwise_and(off, 127)
            two = th_scr[pl.ds(r0, 2), :]
            rolled = pltpu.roll(two, 128 - l0, 1)
            return jnp.where(lane2 < 128 - l0, rolled[0:1, :], rolled[1:2, :])

        def window_body(u, carry):
            r = 8 * u
            w_scr[pl.ds(r, 8), :] = jnp.concatenate(
                [window_row(r + v) for v in range(8)], axis=0)
            return carry

        jax.lax.fori_loop(0, _GPAD // 8, window_body, 0)

        c_grid = jnp.cos(w_scr[:, :])
        s_grid = jnp.sin(w_scr[:, :])
        a = act_ref[:, :]
        l = left_ref[:, :]
        cs_scr[:, :] = a * c_grid + (1.0 - a)
        cl_scr[:, :] = -l * s_grid
        cr_scr[:, :] = (a - l) * s_grid

        rows = jax.lax.broadcasted_iota(jnp.int32, (_NCHUNK, _N, _N), 1)
        cols = jax.lax.broadcasted_iota(jnp.int32, (_NCHUNK, _N, _N), 2)
        qa_scr[:, :, :] = (rows == cols).astype(jnp.float32)

        def apply_layer(q, i):
            coef_self = cs_scr[pl.ds(_NCHUNK * i, _NCHUNK), :].reshape(_NCHUNK, 1, _N)
            coef_l = cl_scr[pl.ds(_NCHUNK * i, _NCHUNK), :].reshape(_NCHUNK, 1, _N)
            coef_r = cr_scr[pl.ds(_NCHUNK * i, _NCHUNK), :].reshape(_NCHUNK, 1, _N)
            q_left = pltpu.roll(q, _N - 1, 2)
            q_right = pltpu.roll(q, 1, 2)
            return coef_self * q + coef_l * q_left + coef_r * q_right

        def body(u, carry):
            q = qa_scr[:, :, :]
            q = apply_layer(q, 4 * u)
            q = apply_layer(q, 4 * u + 1)
            q = apply_layer(q, 4 * u + 2)
            q = apply_layer(q, 4 * u + 3)
            qa_scr[:, :, :] = q
            return carry

        jax.lax.fori_loop(0, _CLEN // 4, body, 0)

        hi = jax.lax.Precision.HIGHEST
        for j in range(4):
            qb_scr[j, :, :] = jnp.dot(
                qa_scr[2 * j, :, :], qa_scr[2 * j + 1, :, :],
                precision=hi, preferred_element_type=jnp.float32)
        for j in range(2):
            qa_scr[j, :, :] = jnp.dot(
                qb_scr[2 * j, :, :], qb_scr[2 * j + 1, :, :],
                precision=hi, preferred_element_type=jnp.float32)
        q_scr[:, :] = jnp.dot(
            qa_scr[0, :, :], qa_scr[1, :, :],
            precision=hi, preferred_element_type=jnp.float32)

    out_ref[:, :] = (
        jnp.dot(x_ref[:, :], q_scr[:, :], preferred_element_type=jnp.float32)
        + bias_ref[0:1, :]
    )


def kernel(inputs, thetas, bias):
    batch = inputs.shape[0]
    th_pad = jnp.pad(thetas, (0, _TROWS * _N - thetas.shape[0])).reshape(_TROWS, _N)
    bias2d = bias.reshape(1, _N)
    grid = batch // _BATCH_TILE
    return pl.pallas_call(
        _kernel_body,
        out_shape=jax.ShapeDtypeStruct((batch, _N), jnp.float32),
        grid=(grid,),
        in_specs=[
            pl.BlockSpec(memory_space=pltpu.SMEM),
            pl.BlockSpec((_TROWS, _N), lambda i: (0, 0)),
            pl.BlockSpec((_GPAD, _N), lambda i: (0, 0)),
            pl.BlockSpec((_GPAD, _N), lambda i: (0, 0)),
            pl.BlockSpec((_BATCH_TILE, _N), lambda i: (i, 0)),
            pl.BlockSpec((1, _N), lambda i: (0, 0)),
        ],
        out_specs=pl.BlockSpec((_BATCH_TILE, _N), lambda i: (i, 0)),
        scratch_shapes=[
            pltpu.VMEM((_N, _N), jnp.float32),
            pltpu.VMEM((_GPAD, _N), jnp.float32),
            pltpu.VMEM((_GPAD, _N), jnp.float32),
            pltpu.VMEM((_GPAD, _N), jnp.float32),
            pltpu.VMEM((_GPAD, _N), jnp.float32),
            pltpu.VMEM((_NCHUNK, _N, _N), jnp.float32),
            pltpu.VMEM((_NCHUNK // 2, _N, _N), jnp.float32),
            pltpu.VMEM((_N, _N), jnp.float32),
        ],
        compiler_params=pltpu.CompilerParams(
            dimension_semantics=("arbitrary",),
        ),
    )(jnp.asarray(_OFFS), th_pad, jnp.asarray(_ACT), jnp.asarray(_LEFT),
      inputs, bias2d)


# speedup vs baseline: 1.2810x; 1.0731x over previous
import numpy as np
import jax
import jax.numpy as jnp
from jax.experimental import pallas as pl
from jax.experimental.pallas import tpu as pltpu

_N = 128
_BATCH_TILE = 8192
_NCHUNK = 8
_CLEN = 32
_GPAD = _NCHUNK * _CLEN
_TROWS = 64


def _static_grids():
    list_wires = [(j - 1, j) for i in range(1, _N) for j in range(i, max(0, i - _N), -1)]
    pos = {}
    groups = [[]]
    for wires in list_wires:
        g_pos = max(pos.get(w, 0) for w in wires)
        while len(groups) - 1 < g_pos:
            groups.append([])
        groups[g_pos].append(wires)
        for w in wires:
            pos[w] = g_pos + 1
    G = len(groups)
    offs = np.zeros((_GPAD,), np.int32)
    act = np.zeros((_GPAD, _N), np.float32)
    left = np.zeros((_GPAD, _N), np.float32)
    layer_off = np.zeros((G,), np.int64)
    layer_pairs = [None] * G
    off = 0
    for g, grp in enumerate(groups):
        layer_off[g] = off
        layer_pairs[g] = grp
        off += len(grp)
    for c in range(_NCHUNK):
        for i in range(_CLEN):
            g = _CLEN * c + i
            r = _NCHUNK * i + c
            if g >= G:
                continue
            offs[r] = max(2 * layer_off[g] - (g % 2), 0)
            for (p0, p1) in layer_pairs[g]:
                act[r, p0] = 1.0
                act[r, p1] = 1.0
                left[r, p0] = 1.0
    return G, offs, act, left


_G, _OFFS, _ACT, _LEFT = _static_grids()


def _kernel_body(offs_ref, th_ref, act_ref, left_ref, x_ref, bias_ref, out_ref,
                 th_scr, w_scr, cs_scr, cl_scr, cr_scr, qa_scr, qb_scr, q_scr):
    @pl.when(pl.program_id(0) == 0)
    def _build_q():
        lane2 = jax.lax.broadcasted_iota(jnp.int32, (1, _N), 1)

        th64 = th_ref[:, :]
        b = th64.reshape(_TROWS, 2, 64).reshape(2 * _TROWS, 64)
        c = jnp.swapaxes(b, 0, 1)
        c2 = jnp.broadcast_to(
            c.reshape(64, 1, 2 * _TROWS), (64, 2, 2 * _TROWS)
        ).reshape(_N, 2 * _TROWS)
        th_scr[:, :] = jnp.swapaxes(c2, 0, 1)

        def window_row(r):
            off = offs_ref[r]
            r0 = jax.lax.shift_right_logical(off, 7)
            l0 = jax.lax.bitwise_and(off, 127)
            two = th_scr[pl.ds(r0, 2), :]
            rolled = pltpu.roll(two, 128 - l0, 1)
            return jnp.where(lane2 < 128 - l0, rolled[0:1, :], rolled[1:2, :])

        def window_body(u, carry):
            r = 16 * u
            w_scr[pl.ds(r, 16), :] = jnp.concatenate(
                [window_row(r + v) for v in range(16)], axis=0)
            return carry

        jax.lax.fori_loop(0, _GPAD // 16, window_body, 0)

        c_grid = jnp.cos(w_scr[:, :])
        s_grid = jnp.sin(w_scr[:, :])
        a = act_ref[:, :]
        l = left_ref[:, :]
        cs_scr[:, :] = a * c_grid + (1.0 - a)
        cl_scr[:, :] = -l * s_grid
        cr_scr[:, :] = (a - l) * s_grid

        rows = jax.lax.broadcasted_iota(jnp.int32, (_NCHUNK, _N, _N), 1)
        cols = jax.lax.broadcasted_iota(jnp.int32, (_NCHUNK, _N, _N), 2)
        qa_scr[:, :, :] = (rows == cols).astype(jnp.float32)

        def apply_layer(q, i):
            coef_self = cs_scr[pl.ds(_NCHUNK * i, _NCHUNK), :].reshape(_NCHUNK, 1, _N)
            coef_l = cl_scr[pl.ds(_NCHUNK * i, _NCHUNK), :].reshape(_NCHUNK, 1, _N)
            coef_r = cr_scr[pl.ds(_NCHUNK * i, _NCHUNK), :].reshape(_NCHUNK, 1, _N)
            q_left = pltpu.roll(q, _N - 1, 2)
            q_right = pltpu.roll(q, 1, 2)
            return coef_self * q + coef_l * q_left + coef_r * q_right

        def body(u, carry):
            q = qa_scr[:, :, :]
            for v in range(8):
                q = apply_layer(q, 8 * u + v)
            qa_scr[:, :, :] = q
            return carry

        jax.lax.fori_loop(0, _CLEN // 8, body, 0)

        hi = jax.lax.Precision.HIGHEST
        for j in range(4):
            qb_scr[j, :, :] = jnp.dot(
                qa_scr[2 * j, :, :], qa_scr[2 * j + 1, :, :],
                precision=hi, preferred_element_type=jnp.float32)
        for j in range(2):
            qa_scr[j, :, :] = jnp.dot(
                qb_scr[2 * j, :, :], qb_scr[2 * j + 1, :, :],
                precision=hi, preferred_element_type=jnp.float32)
        q_scr[:, :] = jnp.dot(
            qa_scr[0, :, :], qa_scr[1, :, :],
            precision=hi, preferred_element_type=jnp.float32)

    out_ref[:, :] = (
        jnp.dot(x_ref[:, :], q_scr[:, :], preferred_element_type=jnp.float32)
        + bias_ref[0:1, :]
    )


def kernel(inputs, thetas, bias):
    batch = inputs.shape[0]
    th_pad = jnp.pad(thetas, (0, _TROWS * _N - thetas.shape[0])).reshape(_TROWS, _N)
    bias2d = bias.reshape(1, _N)
    grid = batch // _BATCH_TILE
    return pl.pallas_call(
        _kernel_body,
        out_shape=jax.ShapeDtypeStruct((batch, _N), jnp.float32),
        grid=(grid,),
        in_specs=[
            pl.BlockSpec(memory_space=pltpu.SMEM),
            pl.BlockSpec((_TROWS, _N), lambda i: (0, 0)),
            pl.BlockSpec((_GPAD, _N), lambda i: (0, 0)),
            pl.BlockSpec((_GPAD, _N), lambda i: (0, 0)),
            pl.BlockSpec((_BATCH_TILE, _N), lambda i: (i, 0)),
            pl.BlockSpec((1, _N), lambda i: (0, 0)),
        ],
        out_specs=pl.BlockSpec((_BATCH_TILE, _N), lambda i: (i, 0)),
        scratch_shapes=[
            pltpu.VMEM((_N, _N), jnp.float32),
            pltpu.VMEM((_GPAD, _N), jnp.float32),
            pltpu.VMEM((_GPAD, _N), jnp.float32),
            pltpu.VMEM((_GPAD, _N), jnp.float32),
            pltpu.VMEM((_GPAD, _N), jnp.float32),
            pltpu.VMEM((_NCHUNK, _N, _N), jnp.float32),
            pltpu.VMEM((_NCHUNK // 2, _N, _N), jnp.float32),
            pltpu.VMEM((_N, _N), jnp.float32),
        ],
        compiler_params=pltpu.CompilerParams(
            dimension_semantics=("arbitrary",),
        ),
    )(jnp.asarray(_OFFS), th_pad, jnp.asarray(_ACT), jnp.asarray(_LEFT),
      inputs, bias2d)


# speedup vs baseline: 1.5879x; 1.2396x over previous
import numpy as np
import jax
import jax.numpy as jnp
from jax.experimental import pallas as pl
from jax.experimental.pallas import tpu as pltpu

_N = 128
_BATCH_TILE = 8192
_NCHUNK = 8
_CLEN = 32
_GPAD = _NCHUNK * _CLEN
_TROWS = 64


def _static_grids():
    list_wires = [(j - 1, j) for i in range(1, _N) for j in range(i, max(0, i - _N), -1)]
    pos = {}
    groups = [[]]
    for wires in list_wires:
        g_pos = max(pos.get(w, 0) for w in wires)
        while len(groups) - 1 < g_pos:
            groups.append([])
        groups[g_pos].append(wires)
        for w in wires:
            pos[w] = g_pos + 1
    G = len(groups)
    offs = np.zeros((_GPAD,), np.int32)
    act = np.zeros((_GPAD, _N), np.float32)
    left = np.zeros((_GPAD, _N), np.float32)
    layer_off = np.zeros((G,), np.int64)
    layer_pairs = [None] * G
    off = 0
    for g, grp in enumerate(groups):
        layer_off[g] = off
        layer_pairs[g] = grp
        off += len(grp)
    for c in range(_NCHUNK):
        for i in range(_CLEN):
            g = _CLEN * c + i
            r = _NCHUNK * i + c
            if g >= G:
                continue
            offs[r] = max(2 * layer_off[g] - (g % 2), 0)
            for (p0, p1) in layer_pairs[g]:
                act[r, p0] = 1.0
                act[r, p1] = 1.0
                left[r, p0] = 1.0
    return G, offs, act, left


_G, _OFFS, _ACT, _LEFT = _static_grids()


def _kernel_body(offs_ref, th_ref, act_ref, left_ref, x_ref, bias_ref, out_ref,
                 th_scr, w_scr, cs_scr, cp_scr, qa_scr, qb_scr, q_scr):
    @pl.when(pl.program_id(0) == 0)
    def _build_q():
        lane2 = jax.lax.broadcasted_iota(jnp.int32, (1, _N), 1)

        th64 = th_ref[:, :]
        b = th64.reshape(_TROWS, 2, 64).reshape(2 * _TROWS, 64)
        c = jnp.swapaxes(b, 0, 1)
        c2 = jnp.broadcast_to(
            c.reshape(64, 1, 2 * _TROWS), (64, 2, 2 * _TROWS)
        ).reshape(_N, 2 * _TROWS)
        th_scr[:, :] = jnp.swapaxes(c2, 0, 1)

        def window_row(r):
            off = offs_ref[r]
            r0 = jax.lax.shift_right_logical(off, 7)
            l0 = jax.lax.bitwise_and(off, 127)
            two = th_scr[pl.ds(r0, 2), :]
            rolled = pltpu.roll(two, 128 - l0, 1)
            return jnp.where(lane2 < 128 - l0, rolled[0:1, :], rolled[1:2, :])

        def window_body(u, carry):
            r = 16 * u
            w_scr[pl.ds(r, 16), :] = jnp.concatenate(
                [window_row(r + v) for v in range(16)], axis=0)
            return carry

        jax.lax.fori_loop(0, _GPAD // 16, window_body, 0)

        c_grid = jnp.cos(w_scr[:, :])
        s_grid = jnp.sin(w_scr[:, :])
        a = act_ref[:, :]
        l = left_ref[:, :]
        cs_scr[:, :] = a * c_grid + (1.0 - a)
        cp_scr[:, :] = (a - 2.0 * l) * s_grid

        rows = jax.lax.broadcasted_iota(jnp.int32, (_NCHUNK, _N, _N), 1)
        cols = jax.lax.broadcasted_iota(jnp.int32, (_NCHUNK, _N, _N), 2)
        qa_scr[:, :, :] = (rows == cols).astype(jnp.float32)

        lanes128 = jax.lax.broadcasted_iota(jnp.int32, (_NCHUNK, _N, _N), 2)
        swap_e_idx = jnp.bitwise_xor(lanes128, 1)
        swap_o_idx = jnp.clip(
            jnp.bitwise_xor(lanes128 - 1, 1) + 1, 0, _N - 1)

        def apply_layer(q, i, parity):
            coef_self = cs_scr[pl.ds(_NCHUNK * i, _NCHUNK), :].reshape(_NCHUNK, 1, _N)
            coef_p = cp_scr[pl.ds(_NCHUNK * i, _NCHUNK), :].reshape(_NCHUNK, 1, _N)
            idx = swap_o_idx if parity else swap_e_idx
            q_sw = jnp.take_along_axis(q, idx, axis=2)
            return coef_self * q + coef_p * q_sw

        def body(u, carry):
            q = qa_scr[:, :, :]
            for v in range(8):
                q = apply_layer(q, 8 * u + v, v % 2)
            qa_scr[:, :, :] = q
            return carry

        jax.lax.fori_loop(0, _CLEN // 8, body, 0)

        hi = jax.lax.Precision.HIGHEST
        for j in range(4):
            qb_scr[j, :, :] = jnp.dot(
                qa_scr[2 * j, :, :], qa_scr[2 * j + 1, :, :],
                precision=hi, preferred_element_type=jnp.float32)
        for j in range(2):
            qa_scr[j, :, :] = jnp.dot(
                qb_scr[2 * j, :, :], qb_scr[2 * j + 1, :, :],
                precision=hi, preferred_element_type=jnp.float32)
        q_scr[:, :] = jnp.dot(
            qa_scr[0, :, :], qa_scr[1, :, :],
            precision=hi, preferred_element_type=jnp.float32)

    out_ref[:, :] = (
        jnp.dot(x_ref[:, :], q_scr[:, :], preferred_element_type=jnp.float32)
        + bias_ref[0:1, :]
    )


def kernel(inputs, thetas, bias):
    batch = inputs.shape[0]
    th_pad = jnp.pad(thetas, (0, _TROWS * _N - thetas.shape[0])).reshape(_TROWS, _N)
    bias2d = bias.reshape(1, _N)
    grid = batch // _BATCH_TILE
    return pl.pallas_call(
        _kernel_body,
        out_shape=jax.ShapeDtypeStruct((batch, _N), jnp.float32),
        grid=(grid,),
        in_specs=[
            pl.BlockSpec(memory_space=pltpu.SMEM),
            pl.BlockSpec((_TROWS, _N), lambda i: (0, 0)),
            pl.BlockSpec((_GPAD, _N), lambda i: (0, 0)),
            pl.BlockSpec((_GPAD, _N), lambda i: (0, 0)),
            pl.BlockSpec((_BATCH_TILE, _N), lambda i: (i, 0)),
            pl.BlockSpec((1, _N), lambda i: (0, 0)),
        ],
        out_specs=pl.BlockSpec((_BATCH_TILE, _N), lambda i: (i, 0)),
        scratch_shapes=[
            pltpu.VMEM((_N, _N), jnp.float32),
            pltpu.VMEM((_GPAD, _N), jnp.float32),
            pltpu.VMEM((_GPAD, _N), jnp.float32),
            pltpu.VMEM((_GPAD, _N), jnp.float32),
            pltpu.VMEM((_NCHUNK, _N, _N), jnp.float32),
            pltpu.VMEM((_NCHUNK // 2, _N, _N), jnp.float32),
            pltpu.VMEM((_N, _N), jnp.float32),
        ],
        compiler_params=pltpu.CompilerParams(
            dimension_semantics=("arbitrary",),
        ),
    )(jnp.asarray(_OFFS), th_pad, jnp.asarray(_ACT), jnp.asarray(_LEFT),
      inputs, bias2d)
